# baseline (speedup 1.0000x reference)
import functools

import jax
import jax.numpy as jnp
from jax import lax
from jax.experimental import pallas as pl
from jax.experimental.pallas import tpu as pltpu
from jax.experimental.pallas import tpu_sc as plsc

NC = 2
NS = 16


def _mesh():
    return plsc.VectorSubcoreMesh(
        core_axis_name="c", subcore_axis_name="s", num_cores=NC, num_subcores=NS
    )


def _make_deg(npad, epad):
    nb = epad // (NC * NS) // 128
    rpt = npad // NS

    @functools.partial(
        pl.kernel,
        mesh=_mesh(),
        out_type=jax.ShapeDtypeStruct((NC, npad), jnp.float32),
        scratch_types=[
            pltpu.VMEM((nb * 128 // GB, GB), jnp.int32),
            pltpu.VMEM((GB,), jnp.float32),
            pltpu.VMEM((rpt,), jnp.float32),
            pltpu.VMEM_SHARED((npad,), jnp.float32),
        ],
    )
    def deg_kernel(dst_hbm, out_hbm, didx, ones_v, zbuf, dsh):
        c = lax.axis_index("c")
        s = lax.axis_index("s")
        w = c * NS + s
        pltpu.sync_copy(dst_hbm.at[pl.ds(pl.multiple_of(w * (nb * 128 // GB), 32), nb * 128 // GB)], didx)
        for i in range(GB // 16):
            ones_v[pl.ds(i * 16, 16)] = jnp.full((16,), 1.0, jnp.float32)
        for i in range(rpt // 16):
            zbuf[pl.ds(i * 16, 16)] = jnp.zeros((16,), jnp.float32)
        pltpu.sync_copy(zbuf, dsh.at[pl.ds(s * rpt, rpt)])
        plsc.subcore_barrier()

        def step(j, carry):
            pltpu.sync_copy(ones_v, dsh.at[didx.at[j]], add=True)
            return carry

        lax.fori_loop(0, nb * 128 // GB, step, 0)
        plsc.subcore_barrier()
        pltpu.sync_copy(dsh.at[pl.ds(s * rpt, rpt)], out_hbm.at[c, pl.ds(s * rpt, rpt)])

    return deg_kernel


PB = 8


GB = 32
NBUF = 8


def _edge_phases(tbl, acc, src_hbm, dst_hbm, base, nb, sidx, didx, rows, sems):
    pb2 = PB * 128 // GB

    def phase(p, carry):
        b2 = pl.multiple_of((128 // GB) * (base + p * PB), 32)
        pltpu.sync_copy(src_hbm.at[pl.ds(b2, pb2)], sidx)
        pltpu.sync_copy(dst_hbm.at[pl.ds(b2, pb2)], didx)
        for b in range(NBUF):
            pltpu.async_copy(tbl.at[sidx.at[b]], rows.at[b], sems[b])

        def step(t, carry2):
            j = t * NBUF
            for b in range(NBUF):
                pltpu.make_async_copy(tbl.at[sidx.at[j + b]], rows.at[b], sems[b]).wait()
                pltpu.sync_copy(rows.at[b], acc.at[didx.at[j + b]], add=True)

                @pl.when(j + b + NBUF < pb2)
                def _():
                    pltpu.async_copy(tbl.at[sidx.at[j + b + NBUF]], rows.at[b], sems[b])

            return carry2

        lax.fori_loop(0, pb2 // NBUF, step, 0)
        return carry

    lax.fori_loop(0, nb // PB, phase, 0)


def _make_agg(npad, epad, w):
    nb = epad // NS // 128
    rpt = npad // NS

    @functools.partial(
        pl.kernel,
        mesh=_mesh(),
        out_type=(
            jax.ShapeDtypeStruct((npad, w), jnp.float32),
            jax.ShapeDtypeStruct((npad, w), jnp.float32),
        ),
        scratch_types=[
            pltpu.VMEM((PB * 128 // GB, GB), jnp.int32),
            pltpu.VMEM((PB * 128 // GB, GB), jnp.int32),
            pltpu.VMEM((NBUF, GB, w), jnp.float32),
            pltpu.VMEM_SHARED((npad, w), jnp.float32),
            pltpu.SemaphoreType.DMA,
            pltpu.SemaphoreType.DMA,
            pltpu.SemaphoreType.DMA,
            pltpu.SemaphoreType.DMA,
            pltpu.SemaphoreType.DMA,
            pltpu.SemaphoreType.DMA,
            pltpu.SemaphoreType.DMA,
            pltpu.SemaphoreType.DMA,
        ],
    )
    def agg_kernel(x0, x1, src_hbm, dst_hbm, o0, o1, sidx, didx, rows, acc,
                   sg0, sg1, sg2, sg3, sg4, sg5, sg6, sg7):
        c = lax.axis_index("c")
        s = lax.axis_index("s")

        def run(tbl, out):
            pltpu.sync_copy(tbl.at[pl.ds(s * rpt, rpt)], acc.at[pl.ds(s * rpt, rpt)])
            plsc.subcore_barrier()
            _edge_phases(tbl, acc, src_hbm, dst_hbm, s * nb, nb,
                         sidx, didx, rows, [sg0, sg1, sg2, sg3, sg4, sg5, sg6, sg7])
            plsc.subcore_barrier()
            pltpu.sync_copy(acc.at[pl.ds(s * rpt, rpt)], out.at[pl.ds(s * rpt, rpt)])

        @pl.when(c == 0)
        def _():
            run(x0, o0)

        @pl.when(c == 1)
        def _():
            run(x1, o1)

    return agg_kernel


def _make_agg_edge(npad, epad, w):
    nb = epad // (NC * NS) // 128
    rpt = npad // NS

    @functools.partial(
        pl.kernel,
        mesh=_mesh(),
        out_type=jax.ShapeDtypeStruct((NC, npad, w), jnp.float32),
        scratch_types=[
            pltpu.VMEM((PB * 128 // GB, GB), jnp.int32),
            pltpu.VMEM((PB * 128 // GB, GB), jnp.int32),
            pltpu.VMEM((NBUF, GB, w), jnp.float32),
            pltpu.VMEM_SHARED((npad, w), jnp.float32),
            pltpu.SemaphoreType.DMA,
            pltpu.SemaphoreType.DMA,
            pltpu.SemaphoreType.DMA,
            pltpu.SemaphoreType.DMA,
            pltpu.SemaphoreType.DMA,
            pltpu.SemaphoreType.DMA,
            pltpu.SemaphoreType.DMA,
            pltpu.SemaphoreType.DMA,
        ],
    )
    def agg_kernel(tbl, src_hbm, dst_hbm, out, sidx, didx, rows, acc,
                   sg0, sg1, sg2, sg3, sg4, sg5, sg6, sg7):
        c = lax.axis_index("c")
        s = lax.axis_index("s")
        wk = c * NS + s

        def zrow(i, carry):
            for j in range(w // 16):
                rows[0, i, pl.ds(j * 16, 16)] = jnp.zeros((16,), jnp.float32)
            return carry

        lax.fori_loop(0, GB, zrow, 0)
        for k in range(rpt // GB):
            pltpu.sync_copy(rows.at[0], acc.at[pl.ds(s * rpt + k * GB, GB)])
        plsc.subcore_barrier()
        _edge_phases(tbl, acc, src_hbm, dst_hbm, wk * nb, nb,
                     sidx, didx, rows, [sg0, sg1, sg2, sg3, sg4, sg5, sg6, sg7])
        plsc.subcore_barrier()
        pltpu.sync_copy(acc.at[pl.ds(s * rpt, rpt)], out.at[c, pl.ds(s * rpt, rpt)])

    return agg_kernel


def kernel(x, edge_index, W1, b1, W2, b2):
    n, ind = x.shape
    e = edge_index.shape[1]
    hid = W1.shape[1]
    emb = W2.shape[1]
    half = hid // 2
    qtr = emb // 2

    npad = -(-(n + 1) // 256) * 256
    epad = -(-e // (NC * NS * 128)) * (NC * NS * 128)
    rblk = 1024
    nrb = npad // rblk

    src = edge_index[0].astype(jnp.int32)
    dst = edge_index[1].astype(jnp.int32)
    pad = n + jnp.arange(epad - e, dtype=jnp.int32) % (npad - n)
    srcp = jnp.concatenate([src, pad]).reshape(epad // GB, GB)
    dstp = jnp.concatenate([dst, pad]).reshape(epad // GB, GB)

    def mm1_body(x_ref, w_ref, o_ref):
        o_ref[...] = jnp.dot(x_ref[...], w_ref[...],
                             preferred_element_type=jnp.float32)

    h = pl.pallas_call(
        mm1_body,
        grid=(nrb,),
        in_specs=[
            pl.BlockSpec((rblk, ind), lambda i: (i, 0)),
            pl.BlockSpec((ind, hid), lambda i: (0, 0)),
        ],
        out_specs=pl.BlockSpec((rblk, hid), lambda i: (i, 0)),
        out_shape=jax.ShapeDtypeStruct((npad, hid), jnp.float32),
    )(x, W1)

    degp = _make_deg(npad, epad)(dstp)

    def pre_body(p0_ref, p1_ref, h_ref, dis_ref, x0_ref, x1_ref):
        rb = pl.program_id(0)
        rows = rb * rblk + lax.broadcasted_iota(jnp.int32, (rblk,), 0)
        deg = p0_ref[...] + p1_ref[...] + 1.0
        dis = jnp.where(rows < n, lax.rsqrt(deg), 0.0)
        dis_ref[...] = dis
        live2 = (rb * rblk + lax.broadcasted_iota(jnp.int32, (rblk, 1), 0)) < n
        hb = h_ref[...]
        x0_ref[...] = jnp.where(live2, hb[:, :half] * dis[:, None], 0.0)
        x1_ref[...] = jnp.where(live2, hb[:, half:] * dis[:, None], 0.0)

    dis, x0p, x1p = pl.pallas_call(
        pre_body,
        grid=(nrb,),
        in_specs=[
            pl.BlockSpec((rblk,), lambda i: (i,)),
            pl.BlockSpec((rblk,), lambda i: (i,)),
            pl.BlockSpec((rblk, hid), lambda i: (i, 0)),
        ],
        out_specs=[
            pl.BlockSpec((rblk,), lambda i: (i,)),
            pl.BlockSpec((rblk, half), lambda i: (i, 0)),
            pl.BlockSpec((rblk, half), lambda i: (i, 0)),
        ],
        out_shape=[
            jax.ShapeDtypeStruct((npad,), jnp.float32),
            jax.ShapeDtypeStruct((npad, half), jnp.float32),
            jax.ShapeDtypeStruct((npad, half), jnp.float32),
        ],
    )(degp[0], degp[1], h)

    a0, a1 = _make_agg(npad, epad, half)(x0p, x1p, srcp, dstp)

    def mid_body(a0_ref, a1_ref, dis_ref, b1_ref, w2_ref, m_ref):
        dis = dis_ref[...]
        lo = jnp.maximum(a0_ref[...] * dis[:, None] + b1_ref[0, :half], 0.0)
        hi = jnp.maximum(a1_ref[...] * dis[:, None] + b1_ref[0, half:], 0.0)
        m = (jnp.dot(lo, w2_ref[:half, :], preferred_element_type=jnp.float32)
             + jnp.dot(hi, w2_ref[half:, :], preferred_element_type=jnp.float32))
        m_ref[...] = m * dis[:, None]

    m = pl.pallas_call(
        mid_body,
        grid=(nrb,),
        in_specs=[
            pl.BlockSpec((rblk, half), lambda i: (i, 0)),
            pl.BlockSpec((rblk, half), lambda i: (i, 0)),
            pl.BlockSpec((rblk,), lambda i: (i,)),
            pl.BlockSpec((1, hid), lambda i: (0, 0)),
            pl.BlockSpec((hid, emb), lambda i: (0, 0)),
        ],
        out_specs=pl.BlockSpec((rblk, emb), lambda i: (i, 0)),
        out_shape=jax.ShapeDtypeStruct((npad, emb), jnp.float32),
    )(a0, a1, dis, b1[None, :], W2)

    q = _make_agg_edge(npad, epad, emb)(m, srcp, dstp)

    def post_body(q0_ref, q1_ref, m_ref, dis_ref, b2_ref, o_ref):
        dis = dis_ref[...]
        tot = q0_ref[0] + q1_ref[0] + m_ref[...]
        o_ref[...] = tot * dis[:, None] + b2_ref[0]

    out = pl.pallas_call(
        post_body,
        grid=(nrb,),
        in_specs=[
            pl.BlockSpec((1, rblk, emb), lambda i: (0, i, 0)),
            pl.BlockSpec((1, rblk, emb), lambda i: (1, i, 0)),
            pl.BlockSpec((rblk, emb), lambda i: (i, 0)),
            pl.BlockSpec((rblk,), lambda i: (i,)),
            pl.BlockSpec((1, emb), lambda i: (0, 0)),
        ],
        out_specs=pl.BlockSpec((rblk, emb), lambda i: (i, 0)),
        out_shape=jax.ShapeDtypeStruct((n, emb), jnp.float32),
    )(q, q, m, dis, b2[None, :])

    return out


# speedup vs baseline: 1.0714x; 1.0714x over previous
import functools

import jax
import jax.numpy as jnp
from jax import lax
from jax.experimental import pallas as pl
from jax.experimental.pallas import tpu as pltpu
from jax.experimental.pallas import tpu_sc as plsc

NC = 2
NS = 16


def _mesh():
    return plsc.VectorSubcoreMesh(
        core_axis_name="c", subcore_axis_name="s", num_cores=NC, num_subcores=NS
    )


def _make_deg(npad, epad):
    nb = epad // (NC * NS) // 128
    rpt = npad // NS

    @functools.partial(
        pl.kernel,
        mesh=_mesh(),
        out_type=jax.ShapeDtypeStruct((NC, npad), jnp.float32),
        scratch_types=[
            pltpu.VMEM((2 * nb, 64), jnp.int32),
            pltpu.VMEM((64,), jnp.float32),
            pltpu.VMEM((rpt,), jnp.float32),
            pltpu.VMEM_SHARED((npad,), jnp.float32),
        ],
    )
    def deg_kernel(dst_hbm, out_hbm, didx, ones_v, zbuf, dsh):
        c = lax.axis_index("c")
        s = lax.axis_index("s")
        w = c * NS + s
        pltpu.sync_copy(dst_hbm.at[pl.ds(w * 2 * nb, 2 * nb)], didx)
        for i in range(64 // 16):
            ones_v[pl.ds(i * 16, 16)] = jnp.full((16,), 1.0, jnp.float32)
        for i in range(rpt // 16):
            zbuf[pl.ds(i * 16, 16)] = jnp.zeros((16,), jnp.float32)
        pltpu.sync_copy(zbuf, dsh.at[pl.ds(s * rpt, rpt)])
        plsc.subcore_barrier()

        def step(j, carry):
            pltpu.sync_copy(ones_v, dsh.at[didx.at[j]], add=True)
            return carry

        lax.fori_loop(0, 2 * nb, step, 0)
        plsc.subcore_barrier()
        pltpu.sync_copy(dsh.at[pl.ds(s * rpt, rpt)], out_hbm.at[c, pl.ds(s * rpt, rpt)])

    return deg_kernel


PB = 8


GB = 64
NBUF = 4


def _edge_phases(tbl, acc, src_hbm, dst_hbm, base, nb, sidx, didx, rows, sems):
    pb2 = 2 * PB

    def phase(p, carry):
        b2 = 2 * (base + p * PB)
        pltpu.sync_copy(src_hbm.at[pl.ds(b2, pb2)], sidx)
        pltpu.sync_copy(dst_hbm.at[pl.ds(b2, pb2)], didx)
        for b in range(NBUF):
            pltpu.async_copy(tbl.at[sidx.at[b]], rows.at[b], sems[b])

        def step(t, carry2):
            j = t * NBUF
            for b in range(NBUF):
                pltpu.make_async_copy(tbl.at[sidx.at[j + b]], rows.at[b], sems[b]).wait()
                pltpu.sync_copy(rows.at[b], acc.at[didx.at[j + b]], add=True)

                @pl.when(j + b + NBUF < pb2)
                def _():
                    pltpu.async_copy(tbl.at[sidx.at[j + b + NBUF]], rows.at[b], sems[b])

            return carry2

        lax.fori_loop(0, pb2 // NBUF, step, 0)
        return carry

    lax.fori_loop(0, nb // PB, phase, 0)


def _make_agg(npad, epad, w):
    nb = epad // NS // 128
    rpt = npad // NS

    @functools.partial(
        pl.kernel,
        mesh=_mesh(),
        out_type=(
            jax.ShapeDtypeStruct((npad, w), jnp.float32),
            jax.ShapeDtypeStruct((npad, w), jnp.float32),
        ),
        scratch_types=[
            pltpu.VMEM((2 * PB, GB), jnp.int32),
            pltpu.VMEM((2 * PB, GB), jnp.int32),
            pltpu.VMEM((NBUF, GB, w), jnp.float32),
            pltpu.VMEM_SHARED((npad, w), jnp.float32),
            pltpu.SemaphoreType.DMA,
            pltpu.SemaphoreType.DMA,
            pltpu.SemaphoreType.DMA,
            pltpu.SemaphoreType.DMA,
        ],
    )
    def agg_kernel(x0, x1, src_hbm, dst_hbm, o0, o1, sidx, didx, rows, acc,
                   sg0, sg1, sg2, sg3):
        c = lax.axis_index("c")
        s = lax.axis_index("s")

        def run(tbl, out):
            pltpu.sync_copy(tbl.at[pl.ds(s * rpt, rpt)], acc.at[pl.ds(s * rpt, rpt)])
            plsc.subcore_barrier()
            _edge_phases(tbl, acc, src_hbm, dst_hbm, s * nb, nb,
                         sidx, didx, rows, [sg0, sg1, sg2, sg3])
            plsc.subcore_barrier()
            pltpu.sync_copy(acc.at[pl.ds(s * rpt, rpt)], out.at[pl.ds(s * rpt, rpt)])

        @pl.when(c == 0)
        def _():
            run(x0, o0)

        @pl.when(c == 1)
        def _():
            run(x1, o1)

    return agg_kernel


def _make_agg_edge(npad, epad, w):
    nb = epad // (NC * NS) // 128
    rpt = npad // NS

    @functools.partial(
        pl.kernel,
        mesh=_mesh(),
        out_type=jax.ShapeDtypeStruct((NC, npad, w), jnp.float32),
        scratch_types=[
            pltpu.VMEM((2 * PB, GB), jnp.int32),
            pltpu.VMEM((2 * PB, GB), jnp.int32),
            pltpu.VMEM((NBUF, GB, w), jnp.float32),
            pltpu.VMEM_SHARED((npad, w), jnp.float32),
            pltpu.SemaphoreType.DMA,
            pltpu.SemaphoreType.DMA,
            pltpu.SemaphoreType.DMA,
            pltpu.SemaphoreType.DMA,
        ],
    )
    def agg_kernel(tbl, src_hbm, dst_hbm, out, sidx, didx, rows, acc,
                   sg0, sg1, sg2, sg3):
        c = lax.axis_index("c")
        s = lax.axis_index("s")
        wk = c * NS + s

        def zrow(i, carry):
            for j in range(w // 16):
                rows[0, i, pl.ds(j * 16, 16)] = jnp.zeros((16,), jnp.float32)
            return carry

        lax.fori_loop(0, GB, zrow, 0)
        for k in range(rpt // GB):
            pltpu.sync_copy(rows.at[0], acc.at[pl.ds(s * rpt + k * GB, GB)])
        plsc.subcore_barrier()
        _edge_phases(tbl, acc, src_hbm, dst_hbm, wk * nb, nb,
                     sidx, didx, rows, [sg0, sg1, sg2, sg3])
        plsc.subcore_barrier()
        pltpu.sync_copy(acc.at[pl.ds(s * rpt, rpt)], out.at[c, pl.ds(s * rpt, rpt)])

    return agg_kernel


def kernel(x, edge_index, W1, b1, W2, b2):
    n, ind = x.shape
    e = edge_index.shape[1]
    hid = W1.shape[1]
    emb = W2.shape[1]
    half = hid // 2
    qtr = emb // 2

    npad = -(-(n + 1) // 256) * 256
    epad = -(-e // (NC * NS * 128)) * (NC * NS * 128)
    rblk = 2048
    nrb = npad // rblk

    src = edge_index[0].astype(jnp.int32)
    dst = edge_index[1].astype(jnp.int32)
    pad = n + jnp.arange(epad - e, dtype=jnp.int32) % (npad - n)
    srcp = jnp.concatenate([src, pad]).reshape(epad // GB, GB)
    dstp = jnp.concatenate([dst, pad]).reshape(epad // GB, GB)

    def mm1_body(x_ref, w_ref, o_ref):
        o_ref[...] = jnp.dot(x_ref[...], w_ref[...],
                             preferred_element_type=jnp.float32)

    h = pl.pallas_call(
        mm1_body,
        grid=(nrb,),
        in_specs=[
            pl.BlockSpec((rblk, ind), lambda i: (i, 0)),
            pl.BlockSpec((ind, hid), lambda i: (0, 0)),
        ],
        out_specs=pl.BlockSpec((rblk, hid), lambda i: (i, 0)),
        out_shape=jax.ShapeDtypeStruct((npad, hid), jnp.float32),
    )(x, W1)

    degp = _make_deg(npad, epad)(dstp)

    def pre_body(p0_ref, p1_ref, h_ref, dis_ref, x0_ref, x1_ref):
        rb = pl.program_id(0)
        rows = rb * rblk + lax.broadcasted_iota(jnp.int32, (rblk,), 0)
        deg = p0_ref[...] + p1_ref[...] + 1.0
        dis = jnp.where(rows < n, lax.rsqrt(deg), 0.0)
        dis_ref[...] = dis
        live2 = (rb * rblk + lax.broadcasted_iota(jnp.int32, (rblk, 1), 0)) < n
        hb = h_ref[...]
        x0_ref[...] = jnp.where(live2, hb[:, :half] * dis[:, None], 0.0)
        x1_ref[...] = jnp.where(live2, hb[:, half:] * dis[:, None], 0.0)

    dis, x0p, x1p = pl.pallas_call(
        pre_body,
        grid=(nrb,),
        in_specs=[
            pl.BlockSpec((rblk,), lambda i: (i,)),
            pl.BlockSpec((rblk,), lambda i: (i,)),
            pl.BlockSpec((rblk, hid), lambda i: (i, 0)),
        ],
        out_specs=[
            pl.BlockSpec((rblk,), lambda i: (i,)),
            pl.BlockSpec((rblk, half), lambda i: (i, 0)),
            pl.BlockSpec((rblk, half), lambda i: (i, 0)),
        ],
        out_shape=[
            jax.ShapeDtypeStruct((npad,), jnp.float32),
            jax.ShapeDtypeStruct((npad, half), jnp.float32),
            jax.ShapeDtypeStruct((npad, half), jnp.float32),
        ],
    )(degp[0], degp[1], h)

    a0, a1 = _make_agg(npad, epad, half)(x0p, x1p, srcp, dstp)

    def mid_body(a0_ref, a1_ref, dis_ref, b1_ref, w2_ref, m_ref):
        dis = dis_ref[...]
        lo = jnp.maximum(a0_ref[...] * dis[:, None] + b1_ref[0, :half], 0.0)
        hi = jnp.maximum(a1_ref[...] * dis[:, None] + b1_ref[0, half:], 0.0)
        m = (jnp.dot(lo, w2_ref[:half, :], preferred_element_type=jnp.float32)
             + jnp.dot(hi, w2_ref[half:, :], preferred_element_type=jnp.float32))
        m_ref[...] = m * dis[:, None]

    m = pl.pallas_call(
        mid_body,
        grid=(nrb,),
        in_specs=[
            pl.BlockSpec((rblk, half), lambda i: (i, 0)),
            pl.BlockSpec((rblk, half), lambda i: (i, 0)),
            pl.BlockSpec((rblk,), lambda i: (i,)),
            pl.BlockSpec((1, hid), lambda i: (0, 0)),
            pl.BlockSpec((hid, emb), lambda i: (0, 0)),
        ],
        out_specs=pl.BlockSpec((rblk, emb), lambda i: (i, 0)),
        out_shape=jax.ShapeDtypeStruct((npad, emb), jnp.float32),
    )(a0, a1, dis, b1[None, :], W2)

    q = _make_agg_edge(npad, epad, emb)(m, srcp, dstp)

    def post_body(q0_ref, q1_ref, m_ref, dis_ref, b2_ref, o_ref):
        dis = dis_ref[...]
        tot = q0_ref[0] + q1_ref[0] + m_ref[...]
        o_ref[...] = tot * dis[:, None] + b2_ref[0]

    out = pl.pallas_call(
        post_body,
        grid=(nrb,),
        in_specs=[
            pl.BlockSpec((1, rblk, emb), lambda i: (0, i, 0)),
            pl.BlockSpec((1, rblk, emb), lambda i: (1, i, 0)),
            pl.BlockSpec((rblk, emb), lambda i: (i, 0)),
            pl.BlockSpec((rblk,), lambda i: (i,)),
            pl.BlockSpec((1, emb), lambda i: (0, 0)),
        ],
        out_specs=pl.BlockSpec((rblk, emb), lambda i: (i, 0)),
        out_shape=jax.ShapeDtypeStruct((n, emb), jnp.float32),
    )(q, q, m, dis, b2[None, :])

    return out
